# baseline (device time: 33336 ns/iter reference)
import jax
import jax.numpy as jnp
from jax import lax
from jax.experimental import pallas as pl
from jax.experimental.pallas import tpu as pltpu

N_DEV = 8
N_TOK = 1024
D = 512
H = 1024
N_EXP = 32
E_LOC = 4
CHUNK = N_TOK // N_DEV
NR = 4
W = H // NR


def kernel(x, router_W, route_idx, expert_W):
    def body(
        x_ref,
        rw_ref,
        ridx_ref,
        ew_ref,
        out_ref,
        psend_ref,
        comm_ref,
        send_sems,
        recv_sems,
    ):
        q = lax.axis_index("i")

        xall = x_ref[:, :]
        scores = jnp.dot(xall, rw_ref[:, :], preferred_element_type=jnp.float32)
        m = jnp.max(scores, axis=1, keepdims=True)
        p = jnp.exp(scores - m)
        p = p / jnp.sum(p, axis=1, keepdims=True)
        iota = lax.broadcasted_iota(jnp.int32, (N_TOK, N_EXP), 1)
        r0 = ridx_ref[:, 0:1]
        r1 = ridx_ref[:, 1:2]
        sel = (iota == r0) | (iota == r1)
        psel = jnp.where(sel, p, 0.0)
        gall = psel / jnp.sum(psel, axis=1, keepdims=True)

        xg = []
        for j in range(E_LOC):
            e = q * E_LOC + j
            gj = jnp.sum(jnp.where(iota == e, gall, 0.0), axis=1, keepdims=True)
            xg.append(xall * gj)

        rdmas = []
        for rnd in range(NR):
            acc = jnp.zeros((N_TOK, W), jnp.float32)
            for j in range(E_LOC):
                acc = acc + jnp.dot(
                    xg[j],
                    ew_ref[j, :, pl.ds(rnd * W, W)],
                    preferred_element_type=jnp.float32,
                )
            psend_ref[rnd] = acc.reshape(N_DEV, CHUNK, W).astype(jnp.bfloat16)
            for o in range(1, N_DEV):
                t = lax.rem(q + o, N_DEV)
                rdma = pltpu.make_async_remote_copy(
                    src_ref=psend_ref.at[rnd, t],
                    dst_ref=comm_ref.at[rnd, o - 1],
                    send_sem=send_sems.at[rnd, o - 1],
                    recv_sem=recv_sems.at[rnd, o - 1],
                    device_id=(t,),
                    device_id_type=pl.DeviceIdType.MESH,
                )
                rdma.start()
                rdmas.append(rdma)

        for rnd in range(NR):
            total = psend_ref[rnd, q].astype(jnp.float32)
            for o in range(1, N_DEV):
                rdmas[rnd * (N_DEV - 1) + o - 1].wait()
                total = total + comm_ref[rnd, o - 1].astype(jnp.float32)
            out_ref[:, pl.ds(rnd * W, W)] = total

    return pl.pallas_call(
        body,
        out_shape=jax.ShapeDtypeStruct((CHUNK, H), jnp.float32),
        in_specs=[pl.BlockSpec(memory_space=pltpu.VMEM)] * 4,
        out_specs=pl.BlockSpec(memory_space=pltpu.VMEM),
        scratch_shapes=[
            pltpu.VMEM((NR, N_DEV, CHUNK, W), jnp.bfloat16),
            pltpu.VMEM((NR, N_DEV - 1, CHUNK, W), jnp.bfloat16),
            pltpu.SemaphoreType.DMA((NR, N_DEV - 1)),
            pltpu.SemaphoreType.DMA((NR, N_DEV - 1)),
        ],
    )(x, router_W, route_idx, expert_W)
